# baseline (speedup 1.0000x reference)
import jax
import jax.numpy as jnp
from jax.experimental import pallas as pl
from jax.experimental.pallas import tpu as pltpu

B, S, D = 4, 2048, 4096
SB = 256
NS = S // SB
N = NS * B
KI = 6
KO = 6


def _body(feat, pos, out, fb, ob, pb, sem_i, sem_o, sem_p):
    def in_copy(j):
        s, b = divmod(j, B)
        return pltpu.make_async_copy(
            feat.at[b, pl.ds(s * SB, SB), :], fb.at[j % KI], sem_i.at[j % KI])

    def out_copy(j):
        s, b = divmod(j, B)
        return pltpu.make_async_copy(
            ob.at[j % KO], out.at[b, pl.ds(s * SB, SB), :], sem_o.at[j % KO])

    def pos_copy(s):
        return pltpu.make_async_copy(
            pos.at[pl.ds(s * SB, SB), :], pb.at[s % 2], sem_p.at[s % 2])

    pos_copy(0).start()
    for j in range(KI):
        in_copy(j).start()

    for j in range(N):
        s, b = divmod(j, B)
        if b == 0:
            pos_copy(s).wait()
            if s + 1 < NS:
                pos_copy(s + 1).start()
        in_copy(j).wait()
        if j >= KO:
            out_copy(j - KO).wait()
        ob[j % KO] = fb[j % KI] + pb[s % 2]
        if j + KI < N:
            in_copy(j + KI).start()
        out_copy(j).start()

    for j in range(N - KO, N):
        out_copy(j).wait()


def kernel(feat, pos_emb_weight):
    pos = pos_emb_weight[:S]
    return pl.pallas_call(
        _body,
        in_specs=[
            pl.BlockSpec(memory_space=pl.ANY),
            pl.BlockSpec(memory_space=pl.ANY),
        ],
        out_specs=pl.BlockSpec(memory_space=pl.ANY),
        out_shape=jax.ShapeDtypeStruct((B, S, D), feat.dtype),
        scratch_shapes=[
            pltpu.VMEM((KI, SB, D), jnp.float32),
            pltpu.VMEM((KO, SB, D), jnp.float32),
            pltpu.VMEM((2, SB, D), jnp.float32),
            pltpu.SemaphoreType.DMA((KI,)),
            pltpu.SemaphoreType.DMA((KO,)),
            pltpu.SemaphoreType.DMA((2,)),
        ],
    )(feat, pos)
